# Initial kernel scaffold; baseline (speedup 1.0000x reference)
import jax
import jax.numpy as jnp
from jax.experimental import pallas as pl


def kernel(x, grid):
    raise NotImplementedError("write your pallas kernel here")


# speedup vs baseline: 1.7490x; 1.7490x over previous
import functools

import jax
import jax.numpy as jnp
from jax import lax
from jax.experimental import pallas as pl
from jax.experimental.pallas import tpu as pltpu
from jax.experimental.pallas import tpu_sc as plsc

N = 2
C = 192
H = 224
W = 224
HW = H * W
NPIX = N * HW
NPLANES = N * C

NC = 2
NS = 16
NW = NC * NS
LANES = 16

PIX_PER_TILE = NPIX // NW
PLANES_PER_TILE = NPLANES // NW
P = 3136
NCHUNK = HW // P

_mesh = plsc.VectorSubcoreMesh(core_axis_name="c", subcore_axis_name="s",
                               num_cores=NC, num_subcores=NS)


def _axis_decomp(v, size):
    vc = jnp.clip(v, -8.0, float(size) + 8.0)
    ti = vc.astype(jnp.int32)
    tf = ti.astype(jnp.float32)
    neg = tf > vc
    flo = tf - jnp.where(neg, 1.0, 0.0).astype(jnp.float32)
    i0 = ti - jnp.where(neg, 1, 0).astype(jnp.int32)
    t = vc - flo
    b = jnp.clip(i0, 0, size - 2)
    lo = i0 == -1
    hi = i0 == size - 1
    oob = (i0 < -1) | (i0 > size - 1)
    one = jnp.full((LANES,), 1.0, jnp.float32)
    zero = jnp.full((LANES,), 0.0, jnp.float32)
    s = jnp.where(oob, zero, jnp.where(lo, t, jnp.where(hi, one - t, one)))
    ta = jnp.where(lo, zero, jnp.where(hi, one, t))
    return b, ta, s


@functools.partial(
    pl.kernel,
    out_type=(
        jax.ShapeDtypeStruct((NPIX,), jnp.int32),
        jax.ShapeDtypeStruct((NPIX,), jnp.float32),
        jax.ShapeDtypeStruct((NPIX,), jnp.float32),
        jax.ShapeDtypeStruct((NPIX,), jnp.float32),
    ),
    mesh=_mesh,
    scratch_types=[
        pltpu.VMEM((PIX_PER_TILE,), jnp.float32),
        pltpu.VMEM((PIX_PER_TILE,), jnp.float32),
        pltpu.VMEM((PIX_PER_TILE,), jnp.int32),
        pltpu.VMEM((PIX_PER_TILE,), jnp.float32),
        pltpu.VMEM((PIX_PER_TILE,), jnp.float32),
        pltpu.VMEM((PIX_PER_TILE,), jnp.float32),
    ],
)
def _index_kernel(gx_hbm, gy_hbm, i00_hbm, tx_hbm, ty_hbm, s_hbm,
                  gxc, gyc, i00c, txc, tyc, sc):
    wid = lax.axis_index("s") * NC + lax.axis_index("c")
    base = wid * PIX_PER_TILE
    pltpu.sync_copy(gx_hbm.at[pl.ds(base, PIX_PER_TILE)], gxc)
    pltpu.sync_copy(gy_hbm.at[pl.ds(base, PIX_PER_TILE)], gyc)

    def vbody(v, carry):
        b = v * LANES
        gxv = gxc[pl.ds(b, LANES)]
        gyv = gyc[pl.ds(b, LANES)]
        ix = ((gxv + 1.0) * float(W) - 1.0) * 0.5
        iy = ((gyv + 1.0) * float(H) - 1.0) * 0.5
        bx, tx, sx = _axis_decomp(ix, W)
        by, ty, sy = _axis_decomp(iy, H)
        i00c[pl.ds(b, LANES)] = by * W + bx
        txc[pl.ds(b, LANES)] = tx
        tyc[pl.ds(b, LANES)] = ty
        sc[pl.ds(b, LANES)] = sx * sy
        return carry

    lax.fori_loop(0, PIX_PER_TILE // LANES, vbody, 0)
    pltpu.sync_copy(i00c, i00_hbm.at[pl.ds(base, PIX_PER_TILE)])
    pltpu.sync_copy(txc, tx_hbm.at[pl.ds(base, PIX_PER_TILE)])
    pltpu.sync_copy(tyc, ty_hbm.at[pl.ds(base, PIX_PER_TILE)])
    pltpu.sync_copy(sc, s_hbm.at[pl.ds(base, PIX_PER_TILE)])


@functools.partial(
    pl.kernel,
    out_type=jax.ShapeDtypeStruct((NPLANES * HW,), jnp.float32),
    mesh=_mesh,
    compiler_params=pltpu.CompilerParams(needs_layout_passes=False),
    scratch_types=[
        pltpu.VMEM((HW,), jnp.float32),
        pltpu.VMEM((P,), jnp.int32),
        pltpu.VMEM((P,), jnp.float32),
        pltpu.VMEM((P,), jnp.float32),
        pltpu.VMEM((P,), jnp.float32),
        pltpu.VMEM((P,), jnp.float32),
    ],
)
def _warp_kernel(x_hbm, i00_hbm, tx_hbm, ty_hbm, s_hbm, out_hbm,
                 plane, i00c, txc, tyc, sc, outc):
    wid = lax.axis_index("s") * NC + lax.axis_index("c")
    n = wid // NS
    pix_base = n * HW

    def plane_body(it, carry):
        pidx = wid * PLANES_PER_TILE + it
        pltpu.sync_copy(x_hbm.at[pl.ds(pidx * HW, HW)], plane)

        def chunk_body(j, carry2):
            off = j * P
            pltpu.sync_copy(i00_hbm.at[pl.ds(pix_base + off, P)], i00c)
            pltpu.sync_copy(tx_hbm.at[pl.ds(pix_base + off, P)], txc)
            pltpu.sync_copy(ty_hbm.at[pl.ds(pix_base + off, P)], tyc)
            pltpu.sync_copy(s_hbm.at[pl.ds(pix_base + off, P)], sc)

            def vbody(v, carry3):
                b = v * LANES
                i0 = i00c[pl.ds(b, LANES)]
                txv = txc[pl.ds(b, LANES)]
                tyv = tyc[pl.ds(b, LANES)]
                sv = sc[pl.ds(b, LANES)]
                v00 = plsc.load_gather(plane, [i0])
                v01 = plsc.load_gather(plane, [i0 + 1])
                v10 = plsc.load_gather(plane, [i0 + W])
                v11 = plsc.load_gather(plane, [i0 + W + 1])
                h0 = v00 + txv * (v01 - v00)
                h1 = v10 + txv * (v11 - v10)
                outc[pl.ds(b, LANES)] = sv * (h0 + tyv * (h1 - h0))
                return carry3

            lax.fori_loop(0, P // LANES, vbody, 0)
            pltpu.sync_copy(outc, out_hbm.at[pl.ds(pidx * HW + off, P)])
            return carry2

        lax.fori_loop(0, NCHUNK, chunk_body, 0)
        return carry

    lax.fori_loop(0, PLANES_PER_TILE, plane_body, 0)


def kernel(x, grid):
    gx = grid[..., 0].reshape(NPIX)
    gy = grid[..., 1].reshape(NPIX)
    i00, tx, ty, s = _index_kernel(gx, gy)
    xf = x.reshape(NPLANES * HW)
    out = _warp_kernel(xf, i00, tx, ty, s)
    return out.reshape(N, C, H, W)


# speedup vs baseline: 2.4090x; 1.3774x over previous
import functools

import jax
import jax.numpy as jnp
from jax import lax
from jax.experimental import pallas as pl
from jax.experimental.pallas import tpu as pltpu
from jax.experimental.pallas import tpu_sc as plsc

N = 2
C = 192
H = 224
W = 224
HW = H * W
NPIX = N * HW
NPLANES = N * C

NC = 2
NS = 16
NW = NC * NS
LANES = 16

PLANES_PER_TILE = NPLANES // NW
P = 1568
NCHUNK = HW // P
VPC = P // LANES
CB = 4 * P
TOTAL_CHUNKS = PLANES_PER_TILE * NCHUNK

_mesh = plsc.VectorSubcoreMesh(core_axis_name="c", subcore_axis_name="s",
                               num_cores=NC, num_subcores=NS)


def _axis_decomp(v, size):
    vc = jnp.clip(v, -8.0, float(size) + 8.0)
    ti = vc.astype(jnp.int32)
    tf = ti.astype(jnp.float32)
    neg = tf > vc
    flo = tf - jnp.where(neg, 1.0, 0.0).astype(jnp.float32)
    i0 = ti - jnp.where(neg, 1, 0).astype(jnp.int32)
    t = vc - flo
    b = jnp.clip(i0, 0, size - 2)
    lo = i0 == -1
    hi = i0 == size - 1
    oob = (i0 < -1) | (i0 > size - 1)
    one = jnp.full((LANES,), 1.0, jnp.float32)
    zero = jnp.full((LANES,), 0.0, jnp.float32)
    s = jnp.where(oob, zero, jnp.where(lo, t, jnp.where(hi, one - t, one)))
    ta = jnp.where(lo, zero, jnp.where(hi, one, t))
    return b, ta, s


@functools.partial(
    pl.kernel,
    out_type=jax.ShapeDtypeStruct((N * NCHUNK * CB,), jnp.int32),
    mesh=_mesh,
    compiler_params=pltpu.CompilerParams(needs_layout_passes=False),
    scratch_types=[
        pltpu.VMEM((P,), jnp.float32),
        pltpu.VMEM((P,), jnp.float32),
        pltpu.VMEM((CB,), jnp.int32),
    ],
)
def _index_kernel(gx_hbm, gy_hbm, cb_hbm, gxc, gyc, cbo):
    wid = lax.axis_index("c") * NS + lax.axis_index("s")

    def chunk(cc, carry):
        g = wid * (N * NCHUNK // NW) + cc
        pltpu.sync_copy(gx_hbm.at[pl.ds(g * P, P)], gxc)
        pltpu.sync_copy(gy_hbm.at[pl.ds(g * P, P)], gyc)

        def vbody(v, carry2):
            b = v * LANES
            gxv = gxc[pl.ds(b, LANES)]
            gyv = gyc[pl.ds(b, LANES)]
            ix = ((gxv + 1.0) * float(W) - 1.0) * 0.5
            iy = ((gyv + 1.0) * float(H) - 1.0) * 0.5
            bx, tx, sx = _axis_decomp(ix, W)
            by, ty, sy = _axis_decomp(iy, H)
            cbo[pl.ds(b, LANES)] = by * W + bx
            cbo[pl.ds(P + b, LANES)] = plsc.bitcast(tx, jnp.int32)
            cbo[pl.ds(2 * P + b, LANES)] = plsc.bitcast(ty, jnp.int32)
            cbo[pl.ds(3 * P + b, LANES)] = plsc.bitcast(sx * sy, jnp.int32)
            return carry2

        lax.fori_loop(0, VPC, vbody, 0)
        pltpu.sync_copy(cbo, cb_hbm.at[pl.ds(g * CB, CB)])
        return carry

    lax.fori_loop(0, N * NCHUNK // NW, chunk, 0)


@functools.partial(
    pl.kernel,
    out_type=jax.ShapeDtypeStruct((NPLANES * HW,), jnp.float32),
    mesh=_mesh,
    compiler_params=pltpu.CompilerParams(needs_layout_passes=False),
    scratch_types=[
        pltpu.VMEM((2 * HW,), jnp.float32),
        pltpu.VMEM((2 * CB,), jnp.int32),
        pltpu.VMEM((2 * P,), jnp.float32),
        pltpu.SemaphoreType.DMA((2,)),
        pltpu.SemaphoreType.DMA((2,)),
        pltpu.SemaphoreType.DMA((2,)),
    ],
)
def _warp_kernel(x_hbm, cb_hbm, out_hbm, planes, cbv, outv, sem_p, sem_c, sem_o):
    wid = lax.axis_index("c") * NS + lax.axis_index("s")
    n = wid // NS
    cb_base = n * NCHUNK * CB

    def plane_src(it):
        return x_hbm.at[pl.ds((wid * PLANES_PER_TILE + it) * HW, HW)]

    def cb_src(j):
        return cb_hbm.at[pl.ds(cb_base + j * CB, CB)]

    pltpu.async_copy(plane_src(0), planes.at[pl.ds(0, HW)], sem_p.at[0])
    pltpu.async_copy(cb_src(0), cbv.at[pl.ds(0, CB)], sem_c.at[0])

    def plane_body(it, carry):
        pp = lax.rem(it, 2)
        pltpu.make_async_copy(plane_src(it), planes.at[pl.ds(pp * HW, HW)],
                              sem_p.at[pp]).wait()

        @pl.when(it < PLANES_PER_TILE - 1)
        def _():
            pltpu.async_copy(plane_src(it + 1),
                             planes.at[pl.ds((1 - pp) * HW, HW)],
                             sem_p.at[1 - pp])

        def chunk_body(j, carry2):
            g = it * NCHUNK + j
            cp = lax.rem(g, 2)
            pltpu.make_async_copy(cb_src(j), cbv.at[pl.ds(cp * CB, CB)],
                                  sem_c.at[cp]).wait()

            @pl.when(g < TOTAL_CHUNKS - 1)
            def _():
                jn = lax.rem(j + 1, NCHUNK)
                pltpu.async_copy(cb_src(jn), cbv.at[pl.ds((1 - cp) * CB, CB)],
                                 sem_c.at[1 - cp])

            @pl.when(g >= 2)
            def _():
                pltpu.make_async_copy(outv.at[pl.ds(cp * P, P)],
                                      out_hbm.at[pl.ds(0, P)],
                                      sem_o.at[cp]).wait()

            ibase = cp * CB
            pbase = pp * HW

            def vbody(v, carry3):
                b = v * LANES
                i0 = cbv[pl.ds(ibase + b, LANES)] + pbase
                txv = plsc.bitcast(cbv[pl.ds(ibase + P + b, LANES)], jnp.float32)
                tyv = plsc.bitcast(cbv[pl.ds(ibase + 2 * P + b, LANES)], jnp.float32)
                sv = plsc.bitcast(cbv[pl.ds(ibase + 3 * P + b, LANES)], jnp.float32)
                v00 = plsc.load_gather(planes, [i0])
                v01 = plsc.load_gather(planes, [i0 + 1])
                v10 = plsc.load_gather(planes, [i0 + W])
                v11 = plsc.load_gather(planes, [i0 + W + 1])
                h0 = v00 + txv * (v01 - v00)
                h1 = v10 + txv * (v11 - v10)
                outv[pl.ds(cp * P + b, LANES)] = sv * (h0 + tyv * (h1 - h0))
                return carry3

            lax.fori_loop(0, VPC, vbody, 0)
            pltpu.async_copy(
                outv.at[pl.ds(cp * P, P)],
                out_hbm.at[pl.ds((wid * PLANES_PER_TILE + it) * HW + j * P, P)],
                sem_o.at[cp])
            return carry2

        lax.fori_loop(0, NCHUNK, chunk_body, 0)
        return carry

    lax.fori_loop(0, PLANES_PER_TILE, plane_body, 0)
    pltpu.make_async_copy(outv.at[pl.ds(0, P)], out_hbm.at[pl.ds(0, P)],
                          sem_o.at[0]).wait()
    pltpu.make_async_copy(outv.at[pl.ds(P, P)], out_hbm.at[pl.ds(0, P)],
                          sem_o.at[1]).wait()


def kernel(x, grid):
    gx = grid[..., 0].reshape(NPIX)
    gy = grid[..., 1].reshape(NPIX)
    cb = _index_kernel(gx, gy)
    xf = x.reshape(NPLANES * HW)
    out = _warp_kernel(xf, cb)
    return out.reshape(N, C, H, W)


# speedup vs baseline: 5.0478x; 2.0954x over previous
import functools

import jax
import jax.numpy as jnp
from jax import lax
from jax.experimental import pallas as pl
from jax.experimental.pallas import tpu as pltpu
from jax.experimental.pallas import tpu_sc as plsc

N = 2
C = 192
H = 224
W = 224
HW = H * W
NPIX = N * HW
NPLANES = N * C

NC = 2
NS = 16
NW = NC * NS
LANES = 16

PLANES_PER_TILE = NPLANES // NW
P = 1568
NCHUNK = HW // P
VPC = P // LANES
CB = 4 * P
TOTAL_CHUNKS = PLANES_PER_TILE * NCHUNK
CHUNKS_PER_SUB = NCHUNK // NS

_mesh = plsc.VectorSubcoreMesh(core_axis_name="c", subcore_axis_name="s",
                               num_cores=NC, num_subcores=NS)


def _axis_decomp(v, size):
    vc = jnp.clip(v, -8.0, float(size) + 8.0)
    ti = vc.astype(jnp.int32)
    tf = ti.astype(jnp.float32)
    neg = tf > vc
    flo = tf - jnp.where(neg, 1.0, 0.0).astype(jnp.float32)
    i0 = ti - jnp.where(neg, 1, 0).astype(jnp.int32)
    t = vc - flo
    b = jnp.clip(i0, 0, size - 2)
    lo = i0 == -1
    hi = i0 == size - 1
    oob = (i0 < -1) | (i0 > size - 1)
    one = jnp.full((LANES,), 1.0, jnp.float32)
    zero = jnp.full((LANES,), 0.0, jnp.float32)
    s = jnp.where(oob, zero, jnp.where(lo, t, jnp.where(hi, one - t, one)))
    ta = jnp.where(lo, zero, jnp.where(hi, one, t))
    return b, ta, s


@functools.partial(
    pl.kernel,
    out_type=jax.ShapeDtypeStruct((NPLANES * HW,), jnp.float32),
    mesh=_mesh,
    compiler_params=pltpu.CompilerParams(needs_layout_passes=False),
    scratch_types=[
        pltpu.VMEM((2 * HW,), jnp.float32),
        pltpu.VMEM((2 * CB,), jnp.int32),
        pltpu.VMEM((2 * P,), jnp.float32),
        pltpu.VMEM_SHARED((NCHUNK * CB,), jnp.int32),
        pltpu.SemaphoreType.DMA((2,)),
        pltpu.SemaphoreType.DMA((2,)),
        pltpu.SemaphoreType.DMA((2,)),
    ],
)
def _warp_kernel(grid_hbm, x_hbm, out_hbm, planes, cbv, outv, cbs,
                 sem_p, sem_c, sem_o):
    n = lax.axis_index("c")
    sub = lax.axis_index("s")
    wid = n * NS + sub

    lane = lax.iota(jnp.int32, LANES)

    def p0_chunk(cc, carry):
        j = sub * CHUNKS_PER_SUB + cc
        g = n * NCHUNK + j
        pltpu.sync_copy(grid_hbm.at[pl.ds(g * 2 * P, 2 * P)],
                        cbv.at[pl.ds(CB, 2 * P)])

        @plsc.parallel_loop(0, P, LANES, unroll=7)
        def p0_body(b):
            src = CB + 2 * b + 2 * lane
            gxv = plsc.bitcast(plsc.load_gather(cbv, [src]), jnp.float32)
            gyv = plsc.bitcast(plsc.load_gather(cbv, [src + 1]), jnp.float32)
            ix = ((gxv + 1.0) * float(W) - 1.0) * 0.5
            iy = ((gyv + 1.0) * float(H) - 1.0) * 0.5
            bx, tx, sx = _axis_decomp(ix, W)
            by, ty, sy = _axis_decomp(iy, H)
            cbv[pl.ds(b, LANES)] = by * W + bx
            cbv[pl.ds(P + b, LANES)] = plsc.bitcast(tx, jnp.int32)
            cbv[pl.ds(2 * P + b, LANES)] = plsc.bitcast(ty, jnp.int32)
            cbv[pl.ds(3 * P + b, LANES)] = plsc.bitcast(sx * sy, jnp.int32)

        pltpu.sync_copy(cbv.at[pl.ds(0, CB)], cbs.at[pl.ds(j * CB, CB)])
        return carry

    lax.fori_loop(0, CHUNKS_PER_SUB, p0_chunk, 0)
    plsc.subcore_barrier()

    def plane_src(it):
        return x_hbm.at[pl.ds((wid * PLANES_PER_TILE + it) * HW, HW)]

    def cb_src(j):
        return cbs.at[pl.ds(j * CB, CB)]

    pltpu.async_copy(plane_src(0), planes.at[pl.ds(0, HW)], sem_p.at[0])
    pltpu.async_copy(cb_src(0), cbv.at[pl.ds(0, CB)], sem_c.at[0])

    def plane_body(it, carry):
        pp = lax.rem(it, 2)
        pltpu.make_async_copy(plane_src(it), planes.at[pl.ds(pp * HW, HW)],
                              sem_p.at[pp]).wait()

        @pl.when(it < PLANES_PER_TILE - 1)
        def _():
            pltpu.async_copy(plane_src(it + 1),
                             planes.at[pl.ds((1 - pp) * HW, HW)],
                             sem_p.at[1 - pp])

        def chunk_body(j, carry2):
            g = it * NCHUNK + j
            cp = lax.rem(g, 2)
            pltpu.make_async_copy(cb_src(j), cbv.at[pl.ds(cp * CB, CB)],
                                  sem_c.at[cp]).wait()

            @pl.when(g < TOTAL_CHUNKS - 1)
            def _():
                jn = lax.rem(j + 1, NCHUNK)
                pltpu.async_copy(cb_src(jn), cbv.at[pl.ds((1 - cp) * CB, CB)],
                                 sem_c.at[1 - cp])

            op = lax.rem(g, 2)
            @pl.when(g >= 2)
            def _():
                pltpu.make_async_copy(outv.at[pl.ds(op * P, P)],
                                      out_hbm.at[pl.ds(0, P)],
                                      sem_o.at[op]).wait()

            ibase = cp * CB
            pbase = pp * HW

            @plsc.parallel_loop(0, P, LANES, unroll=7)
            def vbody(b):
                i0 = cbv[pl.ds(ibase + b, LANES)] + pbase
                txv = plsc.bitcast(cbv[pl.ds(ibase + P + b, LANES)], jnp.float32)
                tyv = plsc.bitcast(cbv[pl.ds(ibase + 2 * P + b, LANES)], jnp.float32)
                sv = plsc.bitcast(cbv[pl.ds(ibase + 3 * P + b, LANES)], jnp.float32)
                v00 = plsc.load_gather(planes, [i0])
                v01 = plsc.load_gather(planes, [i0 + 1])
                v10 = plsc.load_gather(planes, [i0 + W])
                v11 = plsc.load_gather(planes, [i0 + W + 1])
                h0 = v00 + txv * (v01 - v00)
                h1 = v10 + txv * (v11 - v10)
                outv[pl.ds(op * P + b, LANES)] = sv * (h0 + tyv * (h1 - h0))

            pltpu.async_copy(
                outv.at[pl.ds(op * P, P)],
                out_hbm.at[pl.ds((wid * PLANES_PER_TILE + it) * HW + j * P, P)],
                sem_o.at[op])
            return carry2

        lax.fori_loop(0, NCHUNK, chunk_body, 0)
        return carry

    lax.fori_loop(0, PLANES_PER_TILE, plane_body, 0)
    pltpu.make_async_copy(outv.at[pl.ds(0, P)], out_hbm.at[pl.ds(0, P)],
                          sem_o.at[0]).wait()
    pltpu.make_async_copy(outv.at[pl.ds(P, P)], out_hbm.at[pl.ds(0, P)],
                          sem_o.at[1]).wait()


def kernel(x, grid):
    grid_i = lax.bitcast_convert_type(grid, jnp.int32).reshape(NPIX * 2)
    xf = x.reshape(NPLANES * HW)
    out = _warp_kernel(grid_i, xf)
    return out.reshape(N, C, H, W)
